# baseline (speedup 1.0000x reference)
import functools

import jax
import jax.numpy as jnp
from jax.experimental import pallas as pl
from jax.experimental.pallas import tpu as pltpu

_IN_DIM = 768
_E_DIM = 64
_NUM_EMB = 256
_BETA = 0.001
_BM = 128


def _dot(a, b):
    return jax.lax.dot_general(a, b, (((1,), (0,)), ((), ())),
                               preferred_element_type=jnp.float32)


def _dot_exact(a, b):
    return jax.lax.dot_general(a, b, (((1,), (0,)), ((), ())),
                               preferred_element_type=jnp.float32,
                               precision=jax.lax.Precision.HIGHEST)


def _dot_t(a, b):
    return jax.lax.dot_general(a, b, (((1,), (1,)), ((), ())),
                               preferred_element_type=jnp.float32)


def _fused_body(n_blocks,
                x_ref, q_ref, w_ref,
                eW0, eb0, eW1, eb1, eW2, eb2, eW3, eb3,
                dW0, db0, dW1, db1, dW2, db2, dW3, db3,
                cb0, cb1, cb2, cb3,
                out_ref, xq_ref, idx_ref, rq_ref, qd_ref):
    i = pl.program_id(0)

    enc = ((eW0, eb0), (eW1, eb1), (eW2, eb2), (eW3, eb3))
    dec = ((dW0, db0), (dW1, db1), (dW2, db2), (dW3, db3))

    def mlp(h, layers):
        n = len(layers)
        for li, (W, b) in enumerate(layers):
            h = _dot(h, W[...]) + b[...]
            if li < n - 1:
                h = jnp.maximum(h, 0.0)
        return h

    x_e = mlp(x_ref[...], enc)
    q_enc = mlp(q_ref[...], enc)

    n1sq = jnp.sum(x_e * x_e, axis=1, keepdims=True)
    n2sq = jnp.sum(q_enc * q_enc, axis=1, keepdims=True)
    dotp = jnp.sum(x_e * q_enc, axis=1, keepdims=True)
    cos = dotp / jnp.maximum(jnp.sqrt(n1sq) * jnp.sqrt(n2sq), 1e-8)
    qd_block = jnp.sum(w_ref[...] * cos)

    residual = x_e
    xq = jnp.zeros_like(x_e)
    rq_block = jnp.float32(0.0)
    idx_cols = []
    for cb_ref in (cb0, cb1, cb2, cb3):
        cb = cb_ref[...]
        cb_sq = jnp.sum(cb * cb, axis=1)[None, :]
        r_sq = jnp.sum(residual * residual, axis=1, keepdims=True)
        scores = (r_sq - 2.0 * _dot_t(residual, cb)) + cb_sq
        m = jnp.min(scores, axis=1, keepdims=True)
        lane = jax.lax.broadcasted_iota(jnp.int32, scores.shape, 1)
        idx2d = jnp.min(jnp.where(scores == m, lane, _NUM_EMB),
                        axis=1, keepdims=True)
        one_hot = (lane == idx2d).astype(jnp.float32)
        qv = _dot_exact(one_hot, cb)
        diff = qv - residual
        rq_block = rq_block + jnp.sum(diff * diff)
        residual = -diff
        xq = xq + qv
        idx_cols.append(idx2d)

    out_ref[...] = mlp(xq, dec)
    xq_ref[...] = xq
    idx_ref[...] = jnp.concatenate(idx_cols, axis=1)

    @pl.when(i == 0)
    def _init():
        rq_ref[...] = jnp.zeros_like(rq_ref)
        qd_ref[...] = jnp.zeros_like(qd_ref)

    rq_ref[...] = rq_ref[...] + rq_block
    qd_ref[...] = qd_ref[...] + qd_block

    @pl.when(i == n_blocks - 1)
    def _finalize():
        b_total = jnp.float32(n_blocks * _BM)
        rq_ref[...] = rq_ref[...] * ((1.0 + _BETA) / (4.0 * b_total * _E_DIM))
        qd_ref[...] = 1.0 - qd_ref[...] / b_total


def kernel(x, q_embs, labels, qd_align_w,
           enc_W0, enc_b0, enc_W1, enc_b1, enc_W2, enc_b2, enc_W3, enc_b3,
           dec_W0, dec_b0, dec_W1, dec_b1, dec_W2, dec_b2, dec_W3, dec_b3,
           cb0, cb1, cb2, cb3):
    B = x.shape[0]
    n_blocks = B // _BM
    enc_Ws = (enc_W0, enc_W1, enc_W2, enc_W3)
    enc_bs = (enc_b0, enc_b1, enc_b2, enc_b3)
    dec_Ws = (dec_W0, dec_W1, dec_W2, dec_W3)
    dec_bs = (dec_b0, dec_b1, dec_b2, dec_b3)

    def batch_spec(d):
        return pl.BlockSpec((_BM, d), lambda i: (i, 0))

    def whole(a):
        return pl.BlockSpec(a.shape, lambda i: (0,) * a.ndim)

    in_specs = [batch_spec(_IN_DIM), batch_spec(_IN_DIM), batch_spec(1)]
    operands = [x, q_embs, qd_align_w.reshape(B, 1)]
    for W, b in zip(enc_Ws, enc_bs):
        operands += [W, b.reshape(1, -1)]
        in_specs += [whole(W), pl.BlockSpec((1, b.shape[0]), lambda i: (0, 0))]
    for W, b in zip(dec_Ws, dec_bs):
        operands += [W, b.reshape(1, -1)]
        in_specs += [whole(W), pl.BlockSpec((1, b.shape[0]), lambda i: (0, 0))]
    for cb in (cb0, cb1, cb2, cb3):
        operands.append(cb)
        in_specs.append(whole(cb))

    scalar_spec = pl.BlockSpec((1, 1), lambda i: (0, 0))
    out_shapes = (
        jax.ShapeDtypeStruct((B, _IN_DIM), jnp.float32),
        jax.ShapeDtypeStruct((B, _E_DIM), jnp.float32),
        jax.ShapeDtypeStruct((B, 4), jnp.int32),
        jax.ShapeDtypeStruct((1, 1), jnp.float32),
        jax.ShapeDtypeStruct((1, 1), jnp.float32),
    )
    out_specs = (
        batch_spec(_IN_DIM),
        batch_spec(_E_DIM),
        batch_spec(4),
        scalar_spec,
        scalar_spec,
    )

    out, x_q, indices, rq, qd = pl.pallas_call(
        functools.partial(_fused_body, n_blocks),
        grid=(n_blocks,),
        in_specs=in_specs,
        out_specs=out_specs,
        out_shape=out_shapes,
        compiler_params=pltpu.CompilerParams(
            vmem_limit_bytes=128 * 1024 * 1024,
        ),
    )(*operands)

    zeros4 = jnp.zeros((4,), jnp.float32)
    return (out, rq[0, 0], indices, x_q, zeros4, zeros4, qd[0, 0])
